# baseline (speedup 1.0000x reference)
import functools

import jax
import jax.numpy as jnp
from jax import lax
from jax.experimental import pallas as pl
from jax.experimental.pallas import tpu as pltpu
from jax.experimental.pallas import tpu_sc as plsc

R_TILE = 256
C_BLK = 2048
KC = 8
NGSEL = 16
NCAND = NGSEL * 16
BIG = 1.0e30
IBIG = 2**30


def _group_body(n_pad, ro_ref, rd_ref, pts_ref, pen_ref, gsel_ref, msc):
    zpad = jnp.zeros((R_TILE, 5), jnp.float32)
    o3 = ro_ref[...]
    rd = rd_ref[...]
    inv = 1.0 / jnp.sqrt(jnp.sum(rd * rd, axis=1, keepdims=True) + 1e-12)
    dn = jnp.concatenate([rd * inv, zpad], axis=1)
    o = jnp.concatenate([o3, zpad], axis=1)
    c = jnp.sum(o * dn, axis=1, keepdims=True)
    o2 = jnp.sum(o * o, axis=1, keepdims=True)
    oc = o2 - c * c
    nblk = n_pad // C_BLK
    dimn = (((1,), (0,)), ((), ()))

    def blk(b, _):
        p = pts_ref[:, pl.ds(b * C_BLK, C_BLK)]
        px = p[0:1, :]
        py = p[1:2, :]
        pz = p[2:3, :]
        a = px * px + py * py + pz * pz + pen_ref[0:1, pl.ds(b * C_BLK, C_BLK)]
        g1 = lax.dot_general(dn, p, dimn, preferred_element_type=jnp.float32, precision=lax.Precision.HIGHEST)
        g2 = lax.dot_general(o, p, dimn, preferred_element_type=jnp.float32, precision=lax.Precision.HIGHEST)
        d2 = (a - g1 * g1) + (oc - 2.0 * (g2 - c * g1))
        m = d2[:, 0:128]
        for k in range(1, C_BLK // 128):
            m = jnp.minimum(m, d2[:, k * 128:(k + 1) * 128])
        msc[:, pl.ds(b * 128, 128)] = m
        return 0

    lax.fori_loop(0, nblk, blk, 0)

    ng = nblk * 128
    M = msc[...]
    glane = lax.broadcasted_iota(jnp.int32, (R_TILE, ng), 1)
    lane128 = lax.broadcasted_iota(jnp.int32, (R_TILE, 128), 1)
    gout = jnp.full((R_TILE, 128), IBIG, jnp.int32)
    for k in range(NGSEL):
        m = jnp.min(M, axis=1, keepdims=True)
        g = jnp.min(jnp.where(M == m, glane, IBIG), axis=1, keepdims=True)
        M = jnp.where(glane == g, BIG, M)
        gout = jnp.where(lane128 == k, g, gout)
    gsel_ref[...] = gout[:, 0:NGSEL]


def _cand_body(n_real, ro_ref, rd_ref, px_ref, py_ref, pz_ref, grep_ref,
               dist_out, t_out, idx_out):
    ox = ro_ref[:, 0:1]
    oy = ro_ref[:, 1:2]
    oz = ro_ref[:, 2:3]
    dx = rd_ref[:, 0:1]
    dy = rd_ref[:, 1:2]
    dz = rd_ref[:, 2:3]
    inv = 1.0 / jnp.sqrt(dx * dx + dy * dy + dz * dz + 1e-12)
    dx = dx * inv
    dy = dy * inv
    dz = dz * inv

    gl = grep_ref[...]
    lane = lax.broadcasted_iota(jnp.int32, (R_TILE, NCAND), 1)
    pid = (gl // 128) * C_BLK + (gl % 128) + (lane % 16) * 128
    px = px_ref[...]
    py = py_ref[...]
    pz = pz_ref[...]
    ax = px - ox
    ay = py - oy
    az = pz - oz
    t = ax * dx + ay * dy + az * dz
    rx = ax - t * dx
    ry = ay - t * dy
    rz = az - t * dz
    d2 = rx * rx + ry * ry + rz * rz
    d = jnp.sqrt(jnp.maximum(d2, 1e-12))
    d = jnp.where(pid < n_real, d, BIG)

    lane128 = lax.broadcasted_iota(jnp.int32, (R_TILE, 128), 1)
    nv = jnp.full((R_TILE, 128), BIG, jnp.float32)
    nt = jnp.zeros((R_TILE, 128), jnp.float32)
    nc = jnp.full((R_TILE, 128), IBIG, jnp.int32)
    for k in range(KC):
        m = jnp.min(d, axis=1, keepdims=True)
        a = jnp.min(jnp.where(d == m, pid, IBIG), axis=1, keepdims=True)
        sel = pid == a
        tk = jnp.sum(jnp.where(sel, t, 0.0), axis=1, keepdims=True)
        d = jnp.where(sel, BIG, d)
        nv = jnp.where(lane128 == k, m, nv)
        nt = jnp.where(lane128 == k, tk, nt)
        nc = jnp.where(lane128 == k, a, nc)
    dist_out[...] = nv[:, 0:KC]
    t_out[...] = nt[:, 0:KC]
    idx_out[...] = nc[:, 0:KC]


def _gather_groups(table, idxf):
    B = idxf.shape[0]
    D = table.shape[1]
    info = plsc.get_sparse_core_info()
    NC, NS = info.num_cores, info.num_subcores
    NW = NC * NS
    nrow = B // 128
    rpw = nrow // NW
    idx2 = idxf.reshape(nrow, 128)
    mesh = plsc.VectorSubcoreMesh(core_axis_name="c", subcore_axis_name="s")

    nb = min(rpw, 4)

    @functools.partial(
        pl.kernel, mesh=mesh,
        out_type=jax.ShapeDtypeStruct((nrow, 128, D), jnp.float32),
        scratch_types=[
            pltpu.VMEM((rpw, 128), jnp.int32),
            pltpu.VMEM((nb, 128, D), jnp.float32),
            pltpu.SemaphoreType.DMA,
        ],
    )
    def gk(table_hbm, idx_hbm, out_hbm, idx_v, rows_v, sem):
        wid = lax.axis_index("s") * NC + lax.axis_index("c")
        base = wid * rpw
        pltpu.sync_copy(idx_hbm.at[pl.ds(base, rpw)], idx_v)
        for h in range(rpw // nb):
            copies = [
                pltpu.async_copy(
                    table_hbm.at[idx_v.at[h * nb + j]], rows_v.at[j], sem)
                for j in range(nb)
            ]
            for c in copies:
                c.wait()
            pltpu.sync_copy(rows_v, out_hbm.at[pl.ds(base + h * nb, nb)])

    return gk(table, idx2).reshape(B, D)


def kernel(points, ray_o, ray_d):
    n_real = points.shape[0]
    n_rays = ray_o.shape[0]
    nblk = (n_real + C_BLK - 1) // C_BLK
    n_pad = nblk * C_BLK
    ng = nblk * 128

    pts_t = jnp.pad(points.T, ((0, 5), (0, n_pad - n_real)))
    pen = jnp.where(jnp.arange(n_pad) < n_real, 0.0, BIG)[None, :]
    pen = pen.astype(jnp.float32)
    pts_pad = jnp.pad(points, ((0, n_pad - n_real), (0, 0)))
    tbl = pts_pad.reshape(nblk, 16, 128, 3).transpose(0, 2, 1, 3).reshape(ng, 48)
    tbl = jnp.pad(tbl, ((0, 0), (0, 128 - 48)))

    grid = (n_rays // R_TILE,)
    gsel = pl.pallas_call(
        functools.partial(_group_body, n_pad),
        grid=grid,
        in_specs=[
            pl.BlockSpec((R_TILE, 3), lambda i: (i, 0)),
            pl.BlockSpec((R_TILE, 3), lambda i: (i, 0)),
            pl.BlockSpec((8, n_pad), lambda i: (0, 0)),
            pl.BlockSpec((1, n_pad), lambda i: (0, 0)),
        ],
        out_specs=pl.BlockSpec((R_TILE, NGSEL), lambda i: (i, 0)),
        out_shape=jax.ShapeDtypeStruct((n_rays, NGSEL), jnp.int32),
        scratch_shapes=[pltpu.VMEM((R_TILE, ng), jnp.float32)],
    )(ray_o, ray_d, pts_t, pen)

    gathered = _gather_groups(tbl, gsel.reshape(n_rays * NGSEL))[:, 0:48]
    pxyz = gathered.reshape(n_rays, NGSEL, 16, 3).transpose(0, 3, 1, 2)
    pxyz = pxyz.reshape(n_rays, 3 * NCAND)
    px = pxyz[:, 0:NCAND]
    py = pxyz[:, NCAND:2 * NCAND]
    pz = pxyz[:, 2 * NCAND:3 * NCAND]
    grep = jnp.broadcast_to(gsel[:, :, None], (n_rays, NGSEL, 16))
    grep = grep.reshape(n_rays, NCAND)

    out_shapes = (
        jax.ShapeDtypeStruct((n_rays, KC), jnp.float32),
        jax.ShapeDtypeStruct((n_rays, KC), jnp.float32),
        jax.ShapeDtypeStruct((n_rays, KC), jnp.int32),
    )
    return pl.pallas_call(
        functools.partial(_cand_body, n_real),
        grid=grid,
        in_specs=[
            pl.BlockSpec((R_TILE, 3), lambda i: (i, 0)),
            pl.BlockSpec((R_TILE, 3), lambda i: (i, 0)),
            pl.BlockSpec((R_TILE, NCAND), lambda i: (i, 0)),
            pl.BlockSpec((R_TILE, NCAND), lambda i: (i, 0)),
            pl.BlockSpec((R_TILE, NCAND), lambda i: (i, 0)),
            pl.BlockSpec((R_TILE, NCAND), lambda i: (i, 0)),
        ],
        out_specs=(
            pl.BlockSpec((R_TILE, KC), lambda i: (i, 0)),
            pl.BlockSpec((R_TILE, KC), lambda i: (i, 0)),
            pl.BlockSpec((R_TILE, KC), lambda i: (i, 0)),
        ),
        out_shape=out_shapes,
    )(ray_o, ray_d, px, py, pz, grep)
